# baseline (speedup 1.0000x reference)
import functools

import jax
import jax.numpy as jnp
from jax.experimental import pallas as pl
from jax.experimental.pallas import tpu as pltpu

D_MODEL = 768
N_EXPERTS = 8
TOP_K = 2
BALANCE_LOSS_WEIGHT = 0.01
CAPACITY_FACTOR = 1.25

BM = 1024


def _router_body(wt_ref, x_ref, logits_ref, idx_ref, prob_ref, loss_ref,
                 cnt_acc, ps_acc, *, n_tokens):
    i = pl.program_id(0)
    nsteps = pl.num_programs(0)

    xb = x_ref[...]
    wt = wt_ref[...]
    logits = jnp.dot(xb, wt, preferred_element_type=jnp.float32)
    logits_ref[...] = logits

    lt = logits.T
    iota = jax.lax.broadcasted_iota(jnp.int32, (N_EXPERTS, BM), 0)
    m1 = jnp.max(lt, axis=0, keepdims=True)
    i1 = jnp.min(jnp.where(lt == m1, iota, N_EXPERTS), axis=0, keepdims=True)
    masked = jnp.where(iota == i1, -jnp.inf, lt)
    m2 = jnp.max(masked, axis=0, keepdims=True)
    i2 = jnp.min(jnp.where(masked == m2, iota, N_EXPERTS), axis=0, keepdims=True)

    t = jnp.exp(m2 - m1)
    denom2 = 1.0 + t
    p1 = 1.0 / denom2
    p2 = t / denom2
    idx_ref[...] = jnp.concatenate([i1, i2], axis=0)
    prob_ref[...] = jnp.concatenate([p1, p2], axis=0)

    e = jnp.exp(lt - m1)
    gp = e / jnp.sum(e, axis=0, keepdims=True)
    ps_blk = jnp.sum(gp, axis=1, keepdims=True)
    cnt_blk = (jnp.sum(jnp.where(iota == i1, 1.0, 0.0), axis=1, keepdims=True)
               + jnp.sum(jnp.where(iota == i2, 1.0, 0.0), axis=1, keepdims=True))

    @pl.when(i == 0)
    def _init():
        cnt_acc[...] = jnp.zeros_like(cnt_acc)
        ps_acc[...] = jnp.zeros_like(ps_acc)

    cnt_acc[:, :1] += cnt_blk
    ps_acc[:, :1] += ps_blk

    @pl.when(i == nsteps - 1)
    def _fin():
        frac = cnt_acc[:, :1] / (n_tokens * TOP_K)
        avg = ps_acc[:, :1] / n_tokens
        loss = jnp.sum(frac * avg) * (N_EXPERTS * BALANCE_LOSS_WEIGHT)
        loss_ref[...] = loss[None, None]


def kernel(x, gate_w):
    b, s, d = x.shape
    n_tokens = b * s
    x_flat = x.reshape(n_tokens, d)
    wt = gate_w.T
    nsteps = n_tokens // BM

    logits, idx_t, prob_t, loss = pl.pallas_call(
        functools.partial(_router_body, n_tokens=n_tokens),
        grid=(nsteps,),
        in_specs=[
            pl.BlockSpec((d, N_EXPERTS), lambda i: (0, 0)),
            pl.BlockSpec((BM, d), lambda i: (i, 0)),
        ],
        out_specs=[
            pl.BlockSpec((BM, N_EXPERTS), lambda i: (i, 0)),
            pl.BlockSpec((TOP_K, BM), lambda i: (0, i)),
            pl.BlockSpec((TOP_K, BM), lambda i: (0, i)),
            pl.BlockSpec((1, 1), lambda i: (0, 0)),
        ],
        out_shape=[
            jax.ShapeDtypeStruct((n_tokens, N_EXPERTS), jnp.float32),
            jax.ShapeDtypeStruct((TOP_K, n_tokens), jnp.int32),
            jax.ShapeDtypeStruct((TOP_K, n_tokens), jnp.float32),
            jax.ShapeDtypeStruct((1, 1), jnp.float32),
        ],
        scratch_shapes=[
            pltpu.VMEM((N_EXPERTS, 128), jnp.float32),
            pltpu.VMEM((N_EXPERTS, 128), jnp.float32),
        ],
    )(wt, x_flat)

    capacity = max(int(b * s * TOP_K / N_EXPERTS * CAPACITY_FACTOR), 4)
    return (idx_t.T.astype(jnp.int64),
            prob_t.T,
            logits,
            loss.reshape(()),
            jnp.asarray(capacity, dtype=jnp.int32))


# speedup vs baseline: 1.1654x; 1.1654x over previous
import functools

import jax
import jax.numpy as jnp
from jax.experimental import pallas as pl
from jax.experimental.pallas import tpu as pltpu

D_MODEL = 768
N_EXPERTS = 8
TOP_K = 2
BALANCE_LOSS_WEIGHT = 0.01
CAPACITY_FACTOR = 1.25

BM = 2048


def _router_body(wt_ref, x_ref, logits_ref, idx_ref, prob_ref, loss_ref,
                 cnt_acc, ps_acc, *, n_tokens):
    i = pl.program_id(0)
    nsteps = pl.num_programs(0)

    xb = x_ref[...]
    wt = wt_ref[...]
    logits = jnp.dot(xb, wt, preferred_element_type=jnp.float32)
    logits_ref[...] = logits

    lt = logits.T
    iota = jax.lax.broadcasted_iota(jnp.int32, (N_EXPERTS, BM), 0)
    m1 = jnp.max(lt, axis=0, keepdims=True)
    i1 = jnp.min(jnp.where(lt == m1, iota, N_EXPERTS), axis=0, keepdims=True)
    masked = jnp.where(iota == i1, -jnp.inf, lt)
    m2 = jnp.max(masked, axis=0, keepdims=True)
    i2 = jnp.min(jnp.where(masked == m2, iota, N_EXPERTS), axis=0, keepdims=True)

    t = jnp.exp(m2 - m1)
    denom2 = 1.0 + t
    p1 = 1.0 / denom2
    p2 = t / denom2
    idx_ref[...] = jnp.concatenate([i1, i2], axis=0)
    prob_ref[...] = jnp.concatenate([p1, p2], axis=0)

    e = jnp.exp(lt - m1)
    gp = e / jnp.sum(e, axis=0, keepdims=True)
    ps_blk = jnp.sum(gp, axis=1, keepdims=True)
    cnt_blk = (jnp.sum(jnp.where(iota == i1, 1.0, 0.0), axis=1, keepdims=True)
               + jnp.sum(jnp.where(iota == i2, 1.0, 0.0), axis=1, keepdims=True))

    @pl.when(i == 0)
    def _init():
        cnt_acc[...] = jnp.zeros_like(cnt_acc)
        ps_acc[...] = jnp.zeros_like(ps_acc)

    cnt_acc[:, :1] += cnt_blk
    ps_acc[:, :1] += ps_blk

    @pl.when(i == nsteps - 1)
    def _fin():
        frac = cnt_acc[:, :1] / (n_tokens * TOP_K)
        avg = ps_acc[:, :1] / n_tokens
        loss = jnp.sum(frac * avg) * (N_EXPERTS * BALANCE_LOSS_WEIGHT)
        loss_ref[...] = loss[None, None]


def kernel(x, gate_w):
    b, s, d = x.shape
    n_tokens = b * s
    x_flat = x.reshape(n_tokens, d)
    wt = gate_w.T
    nsteps = n_tokens // BM

    logits, idx_t, prob_t, loss = pl.pallas_call(
        functools.partial(_router_body, n_tokens=n_tokens),
        grid=(nsteps,),
        in_specs=[
            pl.BlockSpec((d, N_EXPERTS), lambda i: (0, 0)),
            pl.BlockSpec((BM, d), lambda i: (i, 0)),
        ],
        out_specs=[
            pl.BlockSpec((BM, N_EXPERTS), lambda i: (i, 0)),
            pl.BlockSpec((TOP_K, BM), lambda i: (0, i)),
            pl.BlockSpec((TOP_K, BM), lambda i: (0, i)),
            pl.BlockSpec((1, 1), lambda i: (0, 0)),
        ],
        out_shape=[
            jax.ShapeDtypeStruct((n_tokens, N_EXPERTS), jnp.float32),
            jax.ShapeDtypeStruct((TOP_K, n_tokens), jnp.int32),
            jax.ShapeDtypeStruct((TOP_K, n_tokens), jnp.float32),
            jax.ShapeDtypeStruct((1, 1), jnp.float32),
        ],
        scratch_shapes=[
            pltpu.VMEM((N_EXPERTS, 128), jnp.float32),
            pltpu.VMEM((N_EXPERTS, 128), jnp.float32),
        ],
    )(wt, x_flat)

    capacity = max(int(b * s * TOP_K / N_EXPERTS * CAPACITY_FACTOR), 4)
    return (idx_t.T.astype(jnp.int64),
            prob_t.T,
            logits,
            loss.reshape(()),
            jnp.asarray(capacity, dtype=jnp.int32))


# speedup vs baseline: 1.1937x; 1.0242x over previous
import functools

import jax
import jax.numpy as jnp
from jax.experimental import pallas as pl
from jax.experimental.pallas import tpu as pltpu

D_MODEL = 768
N_EXPERTS = 8
TOP_K = 2
BALANCE_LOSS_WEIGHT = 0.01
CAPACITY_FACTOR = 1.25

BM = 4096


def _router_body(wt_ref, x_ref, logits_ref, idx_ref, prob_ref, loss_ref,
                 cnt_acc, ps_acc, *, n_tokens):
    i = pl.program_id(0)
    nsteps = pl.num_programs(0)

    xb = x_ref[...]
    wt = wt_ref[...]
    logits = jnp.dot(xb, wt, preferred_element_type=jnp.float32)
    logits_ref[...] = logits

    lt = logits.T
    iota = jax.lax.broadcasted_iota(jnp.int32, (N_EXPERTS, BM), 0)
    m1 = jnp.max(lt, axis=0, keepdims=True)
    i1 = jnp.min(jnp.where(lt == m1, iota, N_EXPERTS), axis=0, keepdims=True)
    masked = jnp.where(iota == i1, -jnp.inf, lt)
    m2 = jnp.max(masked, axis=0, keepdims=True)
    i2 = jnp.min(jnp.where(masked == m2, iota, N_EXPERTS), axis=0, keepdims=True)

    t = jnp.exp(m2 - m1)
    denom2 = 1.0 + t
    p1 = 1.0 / denom2
    p2 = t / denom2
    idx_ref[...] = jnp.concatenate([i1, i2], axis=0)
    prob_ref[...] = jnp.concatenate([p1, p2], axis=0)

    e = jnp.exp(lt - m1)
    gp = e / jnp.sum(e, axis=0, keepdims=True)
    ps_blk = jnp.sum(gp, axis=1, keepdims=True)
    cnt_blk = (jnp.sum(jnp.where(iota == i1, 1.0, 0.0), axis=1, keepdims=True)
               + jnp.sum(jnp.where(iota == i2, 1.0, 0.0), axis=1, keepdims=True))

    @pl.when(i == 0)
    def _init():
        cnt_acc[...] = jnp.zeros_like(cnt_acc)
        ps_acc[...] = jnp.zeros_like(ps_acc)

    cnt_acc[:, :1] += cnt_blk
    ps_acc[:, :1] += ps_blk

    @pl.when(i == nsteps - 1)
    def _fin():
        frac = cnt_acc[:, :1] / (n_tokens * TOP_K)
        avg = ps_acc[:, :1] / n_tokens
        loss = jnp.sum(frac * avg) * (N_EXPERTS * BALANCE_LOSS_WEIGHT)
        loss_ref[...] = loss[None, None]


def kernel(x, gate_w):
    b, s, d = x.shape
    n_tokens = b * s
    x_flat = x.reshape(n_tokens, d)
    wt = gate_w.T
    nsteps = n_tokens // BM

    logits, idx_t, prob_t, loss = pl.pallas_call(
        functools.partial(_router_body, n_tokens=n_tokens),
        grid=(nsteps,),
        in_specs=[
            pl.BlockSpec((d, N_EXPERTS), lambda i: (0, 0)),
            pl.BlockSpec((BM, d), lambda i: (i, 0)),
        ],
        out_specs=[
            pl.BlockSpec((BM, N_EXPERTS), lambda i: (i, 0)),
            pl.BlockSpec((TOP_K, BM), lambda i: (0, i)),
            pl.BlockSpec((TOP_K, BM), lambda i: (0, i)),
            pl.BlockSpec((1, 1), lambda i: (0, 0)),
        ],
        out_shape=[
            jax.ShapeDtypeStruct((n_tokens, N_EXPERTS), jnp.float32),
            jax.ShapeDtypeStruct((TOP_K, n_tokens), jnp.int32),
            jax.ShapeDtypeStruct((TOP_K, n_tokens), jnp.float32),
            jax.ShapeDtypeStruct((1, 1), jnp.float32),
        ],
        scratch_shapes=[
            pltpu.VMEM((N_EXPERTS, 128), jnp.float32),
            pltpu.VMEM((N_EXPERTS, 128), jnp.float32),
        ],
    )(wt, x_flat)

    capacity = max(int(b * s * TOP_K / N_EXPERTS * CAPACITY_FACTOR), 4)
    return (idx_t.T.astype(jnp.int64),
            prob_t.T,
            logits,
            loss.reshape(()),
            jnp.asarray(capacity, dtype=jnp.int32))
